# baseline (speedup 1.0000x reference)
import functools

import jax
import jax.numpy as jnp
from jax import lax
from jax.experimental import pallas as pl
from jax.experimental.pallas import tpu as pltpu
from jax.experimental.pallas import tpu_sc as plsc

N = 10000
K = 32
CIN = 128
CH = 64
NK = N * K
EPS = 1e-5

GROUP_NODES = 4
GROUP_ROWS = GROUP_NODES * K
NWORKERS = 32
GROUPS_PER_W = 80
NPAD = NWORKERS * GROUPS_PER_W * GROUP_NODES
RING = 4
ROUNDS = GROUPS_PER_W // RING
W_NODES = GROUPS_PER_W * GROUP_NODES
W_IDX = GROUPS_PER_W * GROUP_ROWS

_pcall = functools.partial(pl.pallas_call)


def _sc_body(nbr_hbm, b_hbm, m_hbm, s_hbm, q_hbm,
             idx_v, r0, r1, r2, r3, om_v, os_v, oq_v,
             sem0, sem1, sem2, sem3):
    wid = lax.axis_index("s") * 2 + lax.axis_index("c")
    rows = [r0, r1, r2, r3]
    sems = [sem0, sem1, sem2, sem3]

    pltpu.sync_copy(nbr_hbm.at[pl.ds(wid * W_IDX, W_IDX)], idx_v)

    def fire(slot, off):
        pltpu.async_copy(
            b_hbm.at[idx_v.at[pl.ds(off, GROUP_ROWS)]], rows[slot], sems[slot])

    def wait_rows(slot):
        pltpu.make_async_copy(
            b_hbm.at[pl.ds(0, GROUP_ROWS)], rows[slot], sems[slot]).wait()

    def compute_group(slot, srow):
        rv = rows[slot]

        def nc_body(nc, carry):
            n = nc // (CH // 16)
            c = nc % (CH // 16)
            sl = pl.ds(c * 16, 16)
            v = rv[n * K, sl]
            vmax = v
            vsum = v
            vsq = v * v
            for k in range(1, K):
                v = rv[n * K + k, sl]
                vmax = jnp.maximum(vmax, v)
                vsum = vsum + v
                vsq = vsq + v * v
            om_v[srow + n, sl] = vmax
            os_v[srow + n, sl] = vsum
            oq_v[srow + n, sl] = vsq
            return carry

        lax.fori_loop(0, GROUP_NODES * (CH // 16), nc_body, 0)

    for r in range(RING):
        fire(r, r * GROUP_ROWS)

    def round_body(u, carry):
        nxt = jnp.minimum((u + 1) * RING, GROUPS_PER_W - RING)
        for r in range(RING):
            wait_rows(r)
            compute_group(r, (u * RING + r) * GROUP_NODES)
            fire(r, (nxt + r) * GROUP_ROWS)
        return carry

    lax.fori_loop(0, ROUNDS, round_body, 0)
    for r in range(RING):
        wait_rows(r)

    nbase = wid * W_NODES
    pltpu.sync_copy(om_v, m_hbm.at[pl.ds(nbase, W_NODES)])
    pltpu.sync_copy(os_v, s_hbm.at[pl.ds(nbase, W_NODES)])
    pltpu.sync_copy(oq_v, q_hbm.at[pl.ds(nbase, W_NODES)])


@functools.lru_cache(maxsize=1)
def _sc_gather():
    return functools.partial(
        pl.kernel,
        mesh=plsc.VectorSubcoreMesh(core_axis_name="c", subcore_axis_name="s"),
        compiler_params=pltpu.CompilerParams(use_tc_tiling_on_sc=False),
        out_type=[jax.ShapeDtypeStruct((NPAD, CH), jnp.float32)] * 3,
        scratch_types=[
            pltpu.VMEM((W_IDX,), jnp.int32),
            pltpu.VMEM((GROUP_ROWS, CH), jnp.float32),
            pltpu.VMEM((GROUP_ROWS, CH), jnp.float32),
            pltpu.VMEM((GROUP_ROWS, CH), jnp.float32),
            pltpu.VMEM((GROUP_ROWS, CH), jnp.float32),
            pltpu.VMEM((W_NODES, CH), jnp.float32),
            pltpu.VMEM((W_NODES, CH), jnp.float32),
            pltpu.VMEM((W_NODES, CH), jnp.float32),
            pltpu.SemaphoreType.DMA,
            pltpu.SemaphoreType.DMA,
            pltpu.SemaphoreType.DMA,
            pltpu.SemaphoreType.DMA,
        ],
    )(_sc_body)


def _gather_reduce(nbr_flat, B):
    return _sc_gather()(nbr_flat, B)


def _in_body(x_ref, wab_ref, b_ref, a_ref, bout_ref):
    ab = jnp.dot(x_ref[...], wab_ref[...], preferred_element_type=jnp.float32, precision=jax.lax.Precision.HIGHEST)
    a_ref[...] = ab[:, :CH] + b_ref[...]
    bout_ref[...] = ab[:, CH:]


def _tc_in(x, wab, brow):
    return _pcall(
        _in_body,
        out_shape=[jax.ShapeDtypeStruct((N, CH), jnp.float32)] * 2,
    )(x, wab, brow)


BM = 2000
GRID = N // BM


def _red_body(a_ref, s_ref, q_ref, st_ref):
    i = pl.program_id(0)
    A = a_ref[...]
    S = s_ref[...]
    r = jnp.concatenate([
        jnp.sum(A, axis=0, keepdims=True),
        jnp.sum(A * A, axis=0, keepdims=True),
        jnp.sum(A * S, axis=0, keepdims=True),
        jnp.sum(S, axis=0, keepdims=True),
        jnp.sum(q_ref[...], axis=0, keepdims=True),
    ], axis=0)

    @pl.when(i == 0)
    def _init():
        st_ref[...] = r

    @pl.when(i > 0)
    def _acc():
        st_ref[...] += r


def _tc_reduce(A, S, Q):
    row = pl.BlockSpec((BM, CH), lambda i: (i, 0))
    return _pcall(
        _red_body,
        grid=(GRID,),
        in_specs=[row, row, row],
        out_specs=pl.BlockSpec((5, CH), lambda i: (0, 0)),
        out_shape=jax.ShapeDtypeStruct((5, CH), jnp.float32),
    )(A, S, Q)


def _make_apply_body(has_res, has_next):
    def body(*refs):
        it = iter(refs)
        a_ref = next(it); m_ref = next(it); st_ref = next(it)
        g_ref = next(it); bt_ref = next(it)
        fprev_ref = next(it) if has_res else None
        if has_next:
            wabn_ref = next(it); bn_ref = next(it)
        f_ref = next(it)
        if has_next:
            an_ref = next(it); bnout_ref = next(it)

        st = st_ref[...]
        mean = st[0:1] * (1.0 / N) + st[3:4] * (1.0 / NK)
        eh2 = st[1:2] * (1.0 / N) + st[2:3] * (2.0 / NK) + st[4:5] * (1.0 / NK)
        var = eh2 - mean * mean
        s = g_ref[...] * lax.rsqrt(var + EPS)
        t = bt_ref[...] - mean * s
        f = jnp.maximum(s * (a_ref[...] + m_ref[...]) + t, 0.0)
        if has_res:
            f = f + fprev_ref[...]
        f_ref[...] = f
        if has_next:
            ab = jnp.dot(f, wabn_ref[...], preferred_element_type=jnp.float32,
                         precision=jax.lax.Precision.HIGHEST)
            an_ref[...] = ab[:, :CH] + bn_ref[...]
            bnout_ref[...] = ab[:, CH:]
    return body


def _tc_apply(A, M, S, Q, g, bt, fprev, wabn, bn):
    has_res = fprev is not None
    has_next = wabn is not None
    st = _tc_reduce(A, S, Q)
    row = pl.BlockSpec((BM, CH), lambda i: (i, 0))
    const = lambda shp: pl.BlockSpec(shp, lambda i: (0, 0))
    in_specs = [row, row, const((5, CH)), const((1, CH)), const((1, CH))]
    args = [A, M, st, g, bt]
    out_shape = [jax.ShapeDtypeStruct((N, CH), jnp.float32)]
    out_specs = [row]
    if has_res:
        in_specs.append(row)
        args.append(fprev)
    if has_next:
        in_specs += [const((CH, 2 * CH)), const((1, CH))]
        args += [wabn, bn]
        out_shape += [jax.ShapeDtypeStruct((N, CH), jnp.float32)] * 2
        out_specs += [row, row]
    return _pcall(
        _make_apply_body(has_res, has_next),
        grid=(GRID,),
        in_specs=in_specs,
        out_specs=out_specs,
        out_shape=out_shape,
    )(*args)


def _fus_body(f0_ref, f1_ref, f2_ref, wf_ref, bf_ref, st_ref):
    i = pl.program_id(0)
    feats = jnp.concatenate([f0_ref[...], f1_ref[...], f2_ref[...]], axis=1)
    z = jnp.dot(feats, wf_ref[...], preferred_element_type=jnp.float32)
    z = z + bf_ref[...]
    ps = jnp.sum(z, axis=0, keepdims=True)
    pq = jnp.sum(z * z, axis=0, keepdims=True)
    pm = jnp.max(z, axis=0, keepdims=True)

    @pl.when(i == 0)
    def _init():
        st_ref[0:1, :] = ps
        st_ref[1:2, :] = pq
        st_ref[2:3, :] = pm

    @pl.when(i > 0)
    def _acc():
        st_ref[0:1, :] += ps
        st_ref[1:2, :] += pq
        st_ref[2:3, :] = jnp.maximum(st_ref[2:3, :], pm)


def _tc_fus(f0, f1, f2, wf, bf):
    row = pl.BlockSpec((BM, CH), lambda i: (i, 0))
    return _pcall(
        _fus_body,
        grid=(GRID,),
        in_specs=[row, row, row,
                  pl.BlockSpec((3 * CH, 1024), lambda i: (0, 0)),
                  pl.BlockSpec((1, 1024), lambda i: (0, 0))],
        out_specs=pl.BlockSpec((3, 1024), lambda i: (0, 0)),
        out_shape=jax.ShapeDtypeStruct((3, 1024), jnp.float32),
    )(f0, f1, f2, wf, bf)


def _p1_body(f0_ref, f1_ref, f2_ref, stf_ref, gf_ref, btf_ref,
             wp1f_ref, wp1x_ref, bp1_ref, z1_ref, st_ref):
    i = pl.program_id(0)
    stf = stf_ref[...]
    mean = stf[0:1] * (1.0 / N)
    var = stf[1:2] * (1.0 / N) - mean * mean
    sf = gf_ref[...] * lax.rsqrt(var + EPS)
    tf = btf_ref[...] - mean * sf
    frow = jnp.maximum(sf * stf[2:3] + tf, 0.0)
    crow = jnp.dot(frow, wp1f_ref[...], preferred_element_type=jnp.float32)
    crow = crow + bp1_ref[...]
    feats = jnp.concatenate([f0_ref[...], f1_ref[...], f2_ref[...]], axis=1)
    z1 = jnp.dot(feats, wp1x_ref[...], preferred_element_type=jnp.float32)
    z1 = z1 + crow
    z1_ref[...] = z1
    ps = jnp.sum(z1, axis=0, keepdims=True)
    pq = jnp.sum(z1 * z1, axis=0, keepdims=True)

    @pl.when(i == 0)
    def _init():
        st_ref[0:1, :] = ps
        st_ref[1:2, :] = pq

    @pl.when(i > 0)
    def _acc():
        st_ref[0:1, :] += ps
        st_ref[1:2, :] += pq


def _tc_p1(f0, f1, f2, stf, gf, btf, wp1f, wp1x, bp1):
    row = pl.BlockSpec((BM, CH), lambda i: (i, 0))
    const = lambda shp: pl.BlockSpec(shp, lambda i: (0, 0))
    return _pcall(
        _p1_body,
        grid=(GRID,),
        in_specs=[row, row, row, const((3, 1024)), const((1, 1024)),
                  const((1, 1024)), const((1024, 512)), const((3 * CH, 512)),
                  const((1, 512))],
        out_specs=[pl.BlockSpec((BM, 512), lambda i: (i, 0)),
                   const((2, 512))],
        out_shape=[jax.ShapeDtypeStruct((N, 512), jnp.float32),
                   jax.ShapeDtypeStruct((2, 512), jnp.float32)],
    )(f0, f1, f2, stf, gf, btf, wp1f, wp1x, bp1)


def _make_mid_body(cout):
    def body(z_ref, st_in_ref, g_ref, bt_ref, w_ref, b_ref, zout_ref, st_ref):
        i = pl.program_id(0)
        st = st_in_ref[...]
        mean = st[0:1] * (1.0 / N)
        var = st[1:2] * (1.0 / N) - mean * mean
        s = g_ref[...] * lax.rsqrt(var + EPS)
        t = bt_ref[...] - mean * s
        u = jnp.maximum(s * z_ref[...] + t, 0.0)
        z2 = jnp.dot(u, w_ref[...], preferred_element_type=jnp.float32)
        z2 = z2 + b_ref[...]
        zout_ref[...] = z2
        ps = jnp.sum(z2, axis=0, keepdims=True)
        pq = jnp.sum(z2 * z2, axis=0, keepdims=True)

        @pl.when(i == 0)
        def _init():
            st_ref[0:1, :] = ps
            st_ref[1:2, :] = pq

        @pl.when(i > 0)
        def _acc():
            st_ref[0:1, :] += ps
            st_ref[1:2, :] += pq
    return body


def _tc_mid(z, st_in, g, bt, w, b, cin, cout):
    const = lambda shp: pl.BlockSpec(shp, lambda i: (0, 0))
    return _pcall(
        _make_mid_body(cout),
        grid=(GRID,),
        in_specs=[pl.BlockSpec((BM, cin), lambda i: (i, 0)),
                  const((2, cin)), const((1, cin)), const((1, cin)),
                  const((cin, cout)), const((1, cout))],
        out_specs=[pl.BlockSpec((BM, cout), lambda i: (i, 0)),
                   const((2, cout))],
        out_shape=[jax.ShapeDtypeStruct((N, cout), jnp.float32),
                   jax.ShapeDtypeStruct((2, cout), jnp.float32)],
    )(z, st_in, g, bt, w, b)


def _p3_body(z_ref, st_in_ref, g_ref, bt_ref, w_ref, b_ref, out_ref):
    st = st_in_ref[...]
    mean = st[0:1] * (1.0 / N)
    var = st[1:2] * (1.0 / N) - mean * mean
    s = g_ref[...] * lax.rsqrt(var + EPS)
    t = bt_ref[...] - mean * s
    u = jnp.maximum(s * z_ref[...] + t, 0.0)
    out = jnp.dot(u, w_ref[...], preferred_element_type=jnp.float32)
    out_ref[...] = out + b_ref[...]


def _tc_p3(z, st_in, g, bt, w, b):
    const = lambda shp: pl.BlockSpec(shp, lambda i: (0, 0))
    return _pcall(
        _p3_body,
        grid=(GRID,),
        in_specs=[pl.BlockSpec((BM, 256), lambda i: (i, 0)),
                  const((2, 256)), const((1, 256)), const((1, 256)),
                  const((256, 19)), const((1, 19))],
        out_specs=pl.BlockSpec((BM, 19), lambda i: (i, 0)),
        out_shape=jax.ShapeDtypeStruct((N, 19), jnp.float32),
    )(z, st_in, g, bt, w, b)


def kernel(features, neighbors, W_head, b_head, g_head, bt_head,
           W_b1, b_b1, g_b1, bt_b1, W_b2, b_b2, g_b2, bt_b2,
           W_fus, b_fus, g_fus, bt_fus, W_p1, b_p1, g_p1, bt_p1,
           W_p2, b_p2, g_p2, bt_p2, W_p3, b_p3):
    r = lambda v: v.reshape(1, -1)
    def wab(W, c):
        return jnp.concatenate([W[:c] - W[c:], W[c:]], axis=1)

    wab0 = wab(W_head, CIN)
    wab1 = wab(W_b1, CH)
    wab2 = wab(W_b2, CH)
    nbr_flat = jnp.pad(neighbors.reshape(-1), (0, (NPAD - N) * K))

    A0, B0 = _tc_in(features, wab0, r(b_head))
    M0, S0, Q0 = _gather_reduce(nbr_flat, B0)
    f0, A1, B1 = _tc_apply(A0, M0, S0, Q0, r(g_head), r(bt_head),
                           None, wab1, r(b_b1))
    M1, S1, Q1 = _gather_reduce(nbr_flat, B1)
    f1, A2, B2 = _tc_apply(A1, M1, S1, Q1, r(g_b1), r(bt_b1),
                           f0, wab2, r(b_b2))
    M2, S2, Q2 = _gather_reduce(nbr_flat, B2)
    (f2,) = _tc_apply(A2, M2, S2, Q2, r(g_b2), r(bt_b2), f1, None, None)

    stf = _tc_fus(f0, f1, f2, W_fus, r(b_fus))
    z1, st1 = _tc_p1(f0, f1, f2, stf, r(g_fus), r(bt_fus),
                     W_p1[:1024], W_p1[1024:], r(b_p1))
    z2, st2 = _tc_mid(z1, st1, r(g_p1), r(bt_p1), W_p2, r(b_p2), 512, 256)
    return _tc_p3(z2, st2, r(g_p2), r(bt_p2), W_p3, r(b_p3))


# speedup vs baseline: 2.3803x; 2.3803x over previous
import functools

import jax
import jax.numpy as jnp
from jax import lax
from jax.experimental import pallas as pl
from jax.experimental.pallas import tpu as pltpu
from jax.experimental.pallas import tpu_sc as plsc

N = 10000
K = 32
CIN = 128
CH = 64
NK = N * K
EPS = 1e-5

GROUP_NODES = 4
GROUP_ROWS = GROUP_NODES * K
NWORKERS = 32
GROUPS_PER_W = 80
NPAD = NWORKERS * GROUPS_PER_W * GROUP_NODES
RING = 2
ROUNDS = GROUPS_PER_W // RING
W_NODES = GROUPS_PER_W * GROUP_NODES
W_IDX = GROUPS_PER_W * GROUP_ROWS

_pcall = functools.partial(pl.pallas_call)


def _sc_body(nbr_hbm, b_hbm, m_hbm, s_hbm, q_hbm,
             idx_v, bsh_v, r0, r1, om_v, os_v, oq_v,
             sem0, sem1):
    sid = lax.axis_index("s")
    wid = sid * 2 + lax.axis_index("c")
    rows = [r0, r1]
    sems = [sem0, sem1]

    FILL = N // 16
    pltpu.sync_copy(b_hbm.at[pl.ds(sid * FILL, FILL)],
                    bsh_v.at[pl.ds(sid * FILL, FILL)])
    plsc.subcore_barrier()

    pltpu.sync_copy(nbr_hbm.at[pl.ds(wid * W_IDX, W_IDX)], idx_v)

    def fire(slot, off):
        pltpu.async_copy(
            bsh_v.at[idx_v.at[pl.ds(off, GROUP_ROWS)]], rows[slot], sems[slot])

    def wait_rows(slot):
        pltpu.make_async_copy(
            b_hbm.at[pl.ds(0, GROUP_ROWS)], rows[slot], sems[slot]).wait()

    def compute_group(slot, srow):
        rv = rows[slot]

        def nc_body(nc, carry):
            n = nc // (CH // 16)
            c = nc % (CH // 16)
            sl = pl.ds(c * 16, 16)
            v = rv[n * K, sl]
            vmax = v
            vsum = v
            vsq = v * v
            for k in range(1, K):
                v = rv[n * K + k, sl]
                vmax = jnp.maximum(vmax, v)
                vsum = vsum + v
                vsq = vsq + v * v
            om_v[srow + n, sl] = vmax
            os_v[srow + n, sl] = vsum
            oq_v[srow + n, sl] = vsq
            return carry

        lax.fori_loop(0, GROUP_NODES * (CH // 16), nc_body, 0)

    for r in range(RING):
        fire(r, r * GROUP_ROWS)

    def round_body(u, carry):
        nxt = jnp.minimum((u + 1) * RING, GROUPS_PER_W - RING)
        for r in range(RING):
            wait_rows(r)
            compute_group(r, (u * RING + r) * GROUP_NODES)
            fire(r, (nxt + r) * GROUP_ROWS)
        return carry

    lax.fori_loop(0, ROUNDS, round_body, 0)
    for r in range(RING):
        wait_rows(r)

    nbase = wid * W_NODES
    pltpu.sync_copy(om_v, m_hbm.at[pl.ds(nbase, W_NODES)])
    pltpu.sync_copy(os_v, s_hbm.at[pl.ds(nbase, W_NODES)])
    pltpu.sync_copy(oq_v, q_hbm.at[pl.ds(nbase, W_NODES)])


@functools.lru_cache(maxsize=1)
def _sc_gather():
    return functools.partial(
        pl.kernel,
        mesh=plsc.VectorSubcoreMesh(core_axis_name="c", subcore_axis_name="s"),
        compiler_params=pltpu.CompilerParams(use_tc_tiling_on_sc=False),
        out_type=[jax.ShapeDtypeStruct((NPAD, CH), jnp.float32)] * 3,
        scratch_types=[
            pltpu.VMEM((W_IDX,), jnp.int32),
            pltpu.VMEM_SHARED((N, CH), jnp.float32),
            pltpu.VMEM((GROUP_ROWS, CH), jnp.float32),
            pltpu.VMEM((GROUP_ROWS, CH), jnp.float32),
            pltpu.VMEM((W_NODES, CH), jnp.float32),
            pltpu.VMEM((W_NODES, CH), jnp.float32),
            pltpu.VMEM((W_NODES, CH), jnp.float32),
            pltpu.SemaphoreType.DMA,
            pltpu.SemaphoreType.DMA,
        ],
    )(_sc_body)


def _gather_reduce(nbr_flat, B):
    return _sc_gather()(nbr_flat, B)


def _in_body(x_ref, wab_ref, b_ref, a_ref, bout_ref):
    ab = jnp.dot(x_ref[...], wab_ref[...], preferred_element_type=jnp.float32, precision=jax.lax.Precision.HIGHEST)
    a_ref[...] = ab[:, :CH] + b_ref[...]
    bout_ref[...] = ab[:, CH:]


def _tc_in(x, wab, brow):
    return _pcall(
        _in_body,
        out_shape=[jax.ShapeDtypeStruct((N, CH), jnp.float32)] * 2,
    )(x, wab, brow)


BM = 2000
GRID = N // BM


def _red_body(a_ref, s_ref, q_ref, st_ref):
    i = pl.program_id(0)
    A = a_ref[...]
    S = s_ref[...]
    r = jnp.concatenate([
        jnp.sum(A, axis=0, keepdims=True),
        jnp.sum(A * A, axis=0, keepdims=True),
        jnp.sum(A * S, axis=0, keepdims=True),
        jnp.sum(S, axis=0, keepdims=True),
        jnp.sum(q_ref[...], axis=0, keepdims=True),
    ], axis=0)

    @pl.when(i == 0)
    def _init():
        st_ref[...] = r

    @pl.when(i > 0)
    def _acc():
        st_ref[...] += r


def _tc_reduce(A, S, Q):
    row = pl.BlockSpec((BM, CH), lambda i: (i, 0))
    return _pcall(
        _red_body,
        grid=(GRID,),
        in_specs=[row, row, row],
        out_specs=pl.BlockSpec((5, CH), lambda i: (0, 0)),
        out_shape=jax.ShapeDtypeStruct((5, CH), jnp.float32),
    )(A, S, Q)


def _make_apply_body(has_res, has_next):
    def body(*refs):
        it = iter(refs)
        a_ref = next(it); m_ref = next(it); st_ref = next(it)
        g_ref = next(it); bt_ref = next(it)
        fprev_ref = next(it) if has_res else None
        if has_next:
            wabn_ref = next(it); bn_ref = next(it)
        f_ref = next(it)
        if has_next:
            an_ref = next(it); bnout_ref = next(it)

        st = st_ref[...]
        mean = st[0:1] * (1.0 / N) + st[3:4] * (1.0 / NK)
        eh2 = st[1:2] * (1.0 / N) + st[2:3] * (2.0 / NK) + st[4:5] * (1.0 / NK)
        var = eh2 - mean * mean
        s = g_ref[...] * lax.rsqrt(var + EPS)
        t = bt_ref[...] - mean * s
        f = jnp.maximum(s * (a_ref[...] + m_ref[...]) + t, 0.0)
        if has_res:
            f = f + fprev_ref[...]
        f_ref[...] = f
        if has_next:
            ab = jnp.dot(f, wabn_ref[...], preferred_element_type=jnp.float32,
                         precision=jax.lax.Precision.HIGHEST)
            an_ref[...] = ab[:, :CH] + bn_ref[...]
            bnout_ref[...] = ab[:, CH:]
    return body


def _tc_apply(A, M, S, Q, g, bt, fprev, wabn, bn):
    has_res = fprev is not None
    has_next = wabn is not None
    st = _tc_reduce(A, S, Q)
    row = pl.BlockSpec((BM, CH), lambda i: (i, 0))
    const = lambda shp: pl.BlockSpec(shp, lambda i: (0, 0))
    in_specs = [row, row, const((5, CH)), const((1, CH)), const((1, CH))]
    args = [A, M, st, g, bt]
    out_shape = [jax.ShapeDtypeStruct((N, CH), jnp.float32)]
    out_specs = [row]
    if has_res:
        in_specs.append(row)
        args.append(fprev)
    if has_next:
        in_specs += [const((CH, 2 * CH)), const((1, CH))]
        args += [wabn, bn]
        out_shape += [jax.ShapeDtypeStruct((N, CH), jnp.float32)] * 2
        out_specs += [row, row]
    return _pcall(
        _make_apply_body(has_res, has_next),
        grid=(GRID,),
        in_specs=in_specs,
        out_specs=out_specs,
        out_shape=out_shape,
    )(*args)


def _fus_body(f0_ref, f1_ref, f2_ref, wf_ref, bf_ref, st_ref):
    i = pl.program_id(0)
    feats = jnp.concatenate([f0_ref[...], f1_ref[...], f2_ref[...]], axis=1)
    z = jnp.dot(feats, wf_ref[...], preferred_element_type=jnp.float32)
    z = z + bf_ref[...]
    ps = jnp.sum(z, axis=0, keepdims=True)
    pq = jnp.sum(z * z, axis=0, keepdims=True)
    pm = jnp.max(z, axis=0, keepdims=True)

    @pl.when(i == 0)
    def _init():
        st_ref[0:1, :] = ps
        st_ref[1:2, :] = pq
        st_ref[2:3, :] = pm

    @pl.when(i > 0)
    def _acc():
        st_ref[0:1, :] += ps
        st_ref[1:2, :] += pq
        st_ref[2:3, :] = jnp.maximum(st_ref[2:3, :], pm)


def _tc_fus(f0, f1, f2, wf, bf):
    row = pl.BlockSpec((BM, CH), lambda i: (i, 0))
    return _pcall(
        _fus_body,
        grid=(GRID,),
        in_specs=[row, row, row,
                  pl.BlockSpec((3 * CH, 1024), lambda i: (0, 0)),
                  pl.BlockSpec((1, 1024), lambda i: (0, 0))],
        out_specs=pl.BlockSpec((3, 1024), lambda i: (0, 0)),
        out_shape=jax.ShapeDtypeStruct((3, 1024), jnp.float32),
    )(f0, f1, f2, wf, bf)


def _p1_body(f0_ref, f1_ref, f2_ref, stf_ref, gf_ref, btf_ref,
             wp1f_ref, wp1x_ref, bp1_ref, z1_ref, st_ref):
    i = pl.program_id(0)
    stf = stf_ref[...]
    mean = stf[0:1] * (1.0 / N)
    var = stf[1:2] * (1.0 / N) - mean * mean
    sf = gf_ref[...] * lax.rsqrt(var + EPS)
    tf = btf_ref[...] - mean * sf
    frow = jnp.maximum(sf * stf[2:3] + tf, 0.0)
    crow = jnp.dot(frow, wp1f_ref[...], preferred_element_type=jnp.float32)
    crow = crow + bp1_ref[...]
    feats = jnp.concatenate([f0_ref[...], f1_ref[...], f2_ref[...]], axis=1)
    z1 = jnp.dot(feats, wp1x_ref[...], preferred_element_type=jnp.float32)
    z1 = z1 + crow
    z1_ref[...] = z1
    ps = jnp.sum(z1, axis=0, keepdims=True)
    pq = jnp.sum(z1 * z1, axis=0, keepdims=True)

    @pl.when(i == 0)
    def _init():
        st_ref[0:1, :] = ps
        st_ref[1:2, :] = pq

    @pl.when(i > 0)
    def _acc():
        st_ref[0:1, :] += ps
        st_ref[1:2, :] += pq


def _tc_p1(f0, f1, f2, stf, gf, btf, wp1f, wp1x, bp1):
    row = pl.BlockSpec((BM, CH), lambda i: (i, 0))
    const = lambda shp: pl.BlockSpec(shp, lambda i: (0, 0))
    return _pcall(
        _p1_body,
        grid=(GRID,),
        in_specs=[row, row, row, const((3, 1024)), const((1, 1024)),
                  const((1, 1024)), const((1024, 512)), const((3 * CH, 512)),
                  const((1, 512))],
        out_specs=[pl.BlockSpec((BM, 512), lambda i: (i, 0)),
                   const((2, 512))],
        out_shape=[jax.ShapeDtypeStruct((N, 512), jnp.float32),
                   jax.ShapeDtypeStruct((2, 512), jnp.float32)],
    )(f0, f1, f2, stf, gf, btf, wp1f, wp1x, bp1)


def _make_mid_body(cout):
    def body(z_ref, st_in_ref, g_ref, bt_ref, w_ref, b_ref, zout_ref, st_ref):
        i = pl.program_id(0)
        st = st_in_ref[...]
        mean = st[0:1] * (1.0 / N)
        var = st[1:2] * (1.0 / N) - mean * mean
        s = g_ref[...] * lax.rsqrt(var + EPS)
        t = bt_ref[...] - mean * s
        u = jnp.maximum(s * z_ref[...] + t, 0.0)
        z2 = jnp.dot(u, w_ref[...], preferred_element_type=jnp.float32)
        z2 = z2 + b_ref[...]
        zout_ref[...] = z2
        ps = jnp.sum(z2, axis=0, keepdims=True)
        pq = jnp.sum(z2 * z2, axis=0, keepdims=True)

        @pl.when(i == 0)
        def _init():
            st_ref[0:1, :] = ps
            st_ref[1:2, :] = pq

        @pl.when(i > 0)
        def _acc():
            st_ref[0:1, :] += ps
            st_ref[1:2, :] += pq
    return body


def _tc_mid(z, st_in, g, bt, w, b, cin, cout):
    const = lambda shp: pl.BlockSpec(shp, lambda i: (0, 0))
    return _pcall(
        _make_mid_body(cout),
        grid=(GRID,),
        in_specs=[pl.BlockSpec((BM, cin), lambda i: (i, 0)),
                  const((2, cin)), const((1, cin)), const((1, cin)),
                  const((cin, cout)), const((1, cout))],
        out_specs=[pl.BlockSpec((BM, cout), lambda i: (i, 0)),
                   const((2, cout))],
        out_shape=[jax.ShapeDtypeStruct((N, cout), jnp.float32),
                   jax.ShapeDtypeStruct((2, cout), jnp.float32)],
    )(z, st_in, g, bt, w, b)


def _p3_body(z_ref, st_in_ref, g_ref, bt_ref, w_ref, b_ref, out_ref):
    st = st_in_ref[...]
    mean = st[0:1] * (1.0 / N)
    var = st[1:2] * (1.0 / N) - mean * mean
    s = g_ref[...] * lax.rsqrt(var + EPS)
    t = bt_ref[...] - mean * s
    u = jnp.maximum(s * z_ref[...] + t, 0.0)
    out = jnp.dot(u, w_ref[...], preferred_element_type=jnp.float32)
    out_ref[...] = out + b_ref[...]


def _tc_p3(z, st_in, g, bt, w, b):
    const = lambda shp: pl.BlockSpec(shp, lambda i: (0, 0))
    return _pcall(
        _p3_body,
        grid=(GRID,),
        in_specs=[pl.BlockSpec((BM, 256), lambda i: (i, 0)),
                  const((2, 256)), const((1, 256)), const((1, 256)),
                  const((256, 19)), const((1, 19))],
        out_specs=pl.BlockSpec((BM, 19), lambda i: (i, 0)),
        out_shape=jax.ShapeDtypeStruct((N, 19), jnp.float32),
    )(z, st_in, g, bt, w, b)


def kernel(features, neighbors, W_head, b_head, g_head, bt_head,
           W_b1, b_b1, g_b1, bt_b1, W_b2, b_b2, g_b2, bt_b2,
           W_fus, b_fus, g_fus, bt_fus, W_p1, b_p1, g_p1, bt_p1,
           W_p2, b_p2, g_p2, bt_p2, W_p3, b_p3):
    r = lambda v: v.reshape(1, -1)
    def wab(W, c):
        return jnp.concatenate([W[:c] - W[c:], W[c:]], axis=1)

    wab0 = wab(W_head, CIN)
    wab1 = wab(W_b1, CH)
    wab2 = wab(W_b2, CH)
    nbr_flat = jnp.pad(neighbors.reshape(-1), (0, (NPAD - N) * K))

    A0, B0 = _tc_in(features, wab0, r(b_head))
    M0, S0, Q0 = _gather_reduce(nbr_flat, B0)
    f0, A1, B1 = _tc_apply(A0, M0, S0, Q0, r(g_head), r(bt_head),
                           None, wab1, r(b_b1))
    M1, S1, Q1 = _gather_reduce(nbr_flat, B1)
    f1, A2, B2 = _tc_apply(A1, M1, S1, Q1, r(g_b1), r(bt_b1),
                           f0, wab2, r(b_b2))
    M2, S2, Q2 = _gather_reduce(nbr_flat, B2)
    (f2,) = _tc_apply(A2, M2, S2, Q2, r(g_b2), r(bt_b2), f1, None, None)

    stf = _tc_fus(f0, f1, f2, W_fus, r(b_fus))
    z1, st1 = _tc_p1(f0, f1, f2, stf, r(g_fus), r(bt_fus),
                     W_p1[:1024], W_p1[1024:], r(b_p1))
    z2, st2 = _tc_mid(z1, st1, r(g_p1), r(bt_p1), W_p2, r(b_p2), 512, 256)
    return _tc_p3(z2, st2, r(g_p2), r(bt_p2), W_p3, r(b_p3))
